# baseline (speedup 1.0000x reference)
import jax
import jax.numpy as jnp
from jax.experimental import pallas as pl

_N, _D, _H = 10000, 128, 128
_BM1 = 200
_BM2 = 400

_DN_T = (((1,), (1,)), ((), ()))
_DN = (((1,), (0,)), ((), ()))


def _h0_kernel(x_ref, w_ref, b_ref, o_ref):
    acc = jax.lax.dot_general(x_ref[...], w_ref[...], _DN_T,
                              preferred_element_type=jnp.float32)
    o_ref[...] = jnp.tanh(acc + b_ref[...])


def _epilogue(hp, hn, wp_ref, wn_ref, bp_ref, bn_ref):
    tp = jax.lax.dot_general(hp.astype(jnp.bfloat16), wp_ref[...], _DN_T,
                             preferred_element_type=jnp.float32) + bp_ref[...]
    tn = jax.lax.dot_general(hn.astype(jnp.bfloat16), wn_ref[...], _DN_T,
                             preferred_element_type=jnp.float32) + bn_ref[...]
    return jnp.tanh(tp + tn)


def _layer1_kernel(ap_ref, an_ref, h_ref, wp_ref, wn_ref, bp_ref, bn_ref,
                   o_ref, apbf_ref, anbf_ref):
    h = h_ref[...]
    ap = ap_ref[...].astype(jnp.bfloat16)
    an = an_ref[...].astype(jnp.bfloat16)
    apbf_ref[...] = ap
    anbf_ref[...] = an
    hp = jax.lax.dot_general(ap, h, _DN, preferred_element_type=jnp.float32)
    hn = jax.lax.dot_general(an, h, _DN, preferred_element_type=jnp.float32)
    o_ref[...] = _epilogue(hp, hn, wp_ref, wn_ref, bp_ref, bn_ref)


def _layer2_kernel(ap_ref, an_ref, h_ref, wp_ref, wn_ref, bp_ref, bn_ref, o_ref):
    h = h_ref[...]
    hp = jax.lax.dot_general(ap_ref[...], h, _DN, preferred_element_type=jnp.float32)
    hn = jax.lax.dot_general(an_ref[...], h, _DN, preferred_element_type=jnp.float32)
    o_ref[...] = _epilogue(hp, hn, wp_ref, wn_ref, bp_ref, bn_ref)


def _common_specs(bm):
    return [
        pl.BlockSpec((bm, _N), lambda i: (i, 0)),
        pl.BlockSpec((bm, _N), lambda i: (i, 0)),
        pl.BlockSpec((_N, _H), lambda i: (0, 0)),
        pl.BlockSpec((_H, _H), lambda i: (0, 0)),
        pl.BlockSpec((_H, _H), lambda i: (0, 0)),
        pl.BlockSpec((1, _H), lambda i: (0, 0)),
        pl.BlockSpec((1, _H), lambda i: (0, 0)),
    ]


def _layer1(A_pos, A_neg, h_bf, Wp_bf, bp, Wn_bf, bn):
    return pl.pallas_call(
        _layer1_kernel,
        grid=(_N // _BM1,),
        in_specs=_common_specs(_BM1),
        out_specs=[
            pl.BlockSpec((_BM1, _H), lambda i: (i, 0)),
            pl.BlockSpec((_BM1, _N), lambda i: (i, 0)),
            pl.BlockSpec((_BM1, _N), lambda i: (i, 0)),
        ],
        out_shape=[
            jax.ShapeDtypeStruct((_N, _H), jnp.float32),
            jax.ShapeDtypeStruct((_N, _N), jnp.bfloat16),
            jax.ShapeDtypeStruct((_N, _N), jnp.bfloat16),
        ],
    )(A_pos, A_neg, h_bf, Wp_bf, Wn_bf, bp.reshape(1, _H), bn.reshape(1, _H))


def _layer2(Ap_bf, An_bf, h_bf, Wp_bf, bp, Wn_bf, bn):
    return pl.pallas_call(
        _layer2_kernel,
        grid=(_N // _BM2,),
        in_specs=_common_specs(_BM2),
        out_specs=pl.BlockSpec((_BM2, _H), lambda i: (i, 0)),
        out_shape=jax.ShapeDtypeStruct((_N, _H), jnp.float32),
    )(Ap_bf, An_bf, h_bf, Wp_bf, Wn_bf, bp.reshape(1, _H), bn.reshape(1, _H))


def kernel(x, A_pos, A_neg, W_in, b_in, Wp0, bp0, Wn0, bn0, Wp1, bp1, Wn1, bn1):
    bf = jnp.bfloat16
    h = pl.pallas_call(
        _h0_kernel,
        out_shape=jax.ShapeDtypeStruct((_N, _H), jnp.float32),
    )(x.astype(bf), W_in.astype(bf), b_in.reshape(1, _H))
    h, Ap_bf, An_bf = _layer1(A_pos, A_neg, h.astype(bf),
                              Wp0.astype(bf), bp0, Wn0.astype(bf), bn0)
    h = _layer2(Ap_bf, An_bf, h.astype(bf),
                Wp1.astype(bf), bp1, Wn1.astype(bf), bn1)
    return h


# speedup vs baseline: 1.0508x; 1.0508x over previous
import jax
import jax.numpy as jnp
from jax.experimental import pallas as pl

_N, _D, _H = 10000, 128, 128
_BM = 200

_DN_T = (((1,), (1,)), ((), ()))
_DN = (((1,), (0,)), ((), ()))


def _h0_kernel(x_ref, w_ref, b_ref, o_ref):
    acc = jax.lax.dot_general(x_ref[...], w_ref[...], _DN_T,
                              preferred_element_type=jnp.float32)
    o_ref[...] = jnp.tanh(acc + b_ref[...])


def _layer_kernel(ap_ref, an_ref, h_ref, hbf_ref, wp_ref, wn_ref,
                  bp_ref, bn_ref, o_ref):
    hp = jax.lax.dot_general(ap_ref[...], h_ref[...], _DN,
                             preferred_element_type=jnp.float32)
    hn = jax.lax.dot_general(an_ref[...].astype(jnp.bfloat16), hbf_ref[...],
                             _DN, preferred_element_type=jnp.float32)
    tp = jax.lax.dot_general(hp, wp_ref[...], _DN_T,
                             preferred_element_type=jnp.float32) + bp_ref[...]
    tn = jax.lax.dot_general(hn, wn_ref[...], _DN_T,
                             preferred_element_type=jnp.float32) + bn_ref[...]
    o_ref[...] = jnp.tanh(tp + tn)


def _layer(A_pos, A_neg, h, h_bf, Wp, bp, Wn, bn):
    nb = _N // _BM
    return pl.pallas_call(
        _layer_kernel,
        grid=(nb,),
        in_specs=[
            pl.BlockSpec((_BM, _N), lambda i: (i, 0)),
            pl.BlockSpec((_BM, _N), lambda i: (i, 0)),
            pl.BlockSpec((_N, _H), lambda i: (0, 0)),
            pl.BlockSpec((_N, _H), lambda i: (0, 0)),
            pl.BlockSpec((_H, _H), lambda i: (0, 0)),
            pl.BlockSpec((_H, _H), lambda i: (0, 0)),
            pl.BlockSpec((1, _H), lambda i: (0, 0)),
            pl.BlockSpec((1, _H), lambda i: (0, 0)),
        ],
        out_specs=pl.BlockSpec((_BM, _H), lambda i: (i, 0)),
        out_shape=jax.ShapeDtypeStruct((_N, _H), jnp.float32),
    )(A_pos, A_neg, h, h_bf, Wp, Wn, bp.reshape(1, _H), bn.reshape(1, _H))


def kernel(x, A_pos, A_neg, W_in, b_in, Wp0, bp0, Wn0, bn0, Wp1, bp1, Wn1, bn1):
    h = pl.pallas_call(
        _h0_kernel,
        out_shape=jax.ShapeDtypeStruct((_N, _H), jnp.float32),
    )(x, W_in, b_in.reshape(1, _H))
    bf = jnp.bfloat16
    h = _layer(A_pos, A_neg, h, h.astype(bf), Wp0, bp0, Wn0.astype(bf), bn0)
    h = _layer(A_pos, A_neg, h, h.astype(bf), Wp1, bp1, Wn1.astype(bf), bn1)
    return h
